# baseline (speedup 1.0000x reference)
import functools

import jax
import jax.numpy as jnp
from jax import lax
from jax.experimental import pallas as pl
from jax.experimental.pallas import tpu as pltpu
from jax.experimental.pallas import tpu_sc as plsc

NUM_CORES = 2
NUM_SUBCORES = 16
NUM_WORKERS = NUM_CORES * NUM_SUBCORES
LANES = 16
ROWS = 64
COLS = 32768
ROWS_PER_WORKER = ROWS // NUM_WORKERS
VECS_PER_ROW = COLS // LANES
PROBE = 1024

_MESH = plsc.VectorSubcoreMesh(
    core_axis_name="c",
    subcore_axis_name="s",
    num_cores=NUM_CORES,
    num_subcores=NUM_SUBCORES,
)


def _scan_vectors(buf, n_vecs, acc):
    def body(i, acc):
        j = i * LANES
        a = buf[pl.ds(j, LANES)]
        b = buf[pl.ds(j + 1, LANES)]
        return acc + jnp.where(a > b, 1.0, 0.0)

    return lax.fori_loop(0, n_vecs, body, acc, unroll=4)


@functools.partial(
    pl.kernel,
    out_type=jax.ShapeDtypeStruct((NUM_WORKERS, LANES), jnp.float32),
    mesh=_MESH,
    scratch_types=[
        pltpu.VMEM((PROBE + LANES,), jnp.float32),
        pltpu.VMEM((PROBE + LANES,), jnp.float32),
        pltpu.VMEM((LANES,), jnp.float32),
        pltpu.SemaphoreType.DMA,
        pltpu.SemaphoreType.DMA,
    ],
)
def _probe_check(x_hbm, out_hbm, buf0, buf1, res_v, sem0, sem1):
    wid = lax.axis_index("s") * NUM_CORES + lax.axis_index("c")
    base = wid * (ROWS_PER_WORKER * COLS)
    n = PROBE + LANES
    cp0 = pltpu.async_copy(x_hbm.at[pl.ds(base, n)], buf0, sem0)
    cp1 = pltpu.async_copy(x_hbm.at[pl.ds(base + COLS, n)], buf1, sem1)
    cp0.wait()
    acc = _scan_vectors(buf0, PROBE // LANES, jnp.zeros((LANES,), jnp.float32))
    cp1.wait()
    acc = _scan_vectors(buf1, PROBE // LANES, acc)
    res_v[...] = acc
    pltpu.sync_copy(res_v, out_hbm.at[wid])


@functools.partial(
    pl.kernel,
    out_type=jax.ShapeDtypeStruct((NUM_WORKERS, LANES), jnp.float32),
    mesh=_MESH,
    scratch_types=[
        pltpu.VMEM((COLS + LANES,), jnp.float32),
        pltpu.VMEM((COLS + LANES,), jnp.float32),
        pltpu.VMEM((LANES,), jnp.float32),
        pltpu.SemaphoreType.DMA,
        pltpu.SemaphoreType.DMA,
    ],
)
def _full_check(x_hbm, out_hbm, buf0, buf1, res_v, sem0, sem1):
    wid = lax.axis_index("s") * NUM_CORES + lax.axis_index("c")
    base = wid * (ROWS_PER_WORKER * COLS)
    cp0 = pltpu.async_copy(x_hbm.at[pl.ds(base, COLS)], buf0.at[pl.ds(0, COLS)], sem0)
    cp1 = pltpu.async_copy(x_hbm.at[pl.ds(base + COLS, COLS)], buf1.at[pl.ds(0, COLS)], sem1)
    sentinel = jnp.full((LANES,), jnp.inf, dtype=jnp.float32)
    cp0.wait()
    buf0[pl.ds(COLS, LANES)] = sentinel
    acc = _scan_vectors(buf0, VECS_PER_ROW, jnp.zeros((LANES,), jnp.float32))
    cp1.wait()
    buf1[pl.ds(COLS, LANES)] = sentinel
    acc = _scan_vectors(buf1, VECS_PER_ROW, acc)
    res_v[...] = acc
    pltpu.sync_copy(res_v, out_hbm.at[wid])


def kernel(x):
    xf = x.reshape(-1)
    probe_counts = _probe_check(xf)

    def fast(_):
        return jnp.float32(0.0)

    def slow(xx):
        return (jnp.sum(_full_check(xx)) == 0.0).astype(jnp.float32)

    return lax.cond(jnp.sum(probe_counts) > 0.0, fast, slow, xf)
